# baseline (speedup 1.0000x reference)
import functools

import jax
import jax.numpy as jnp
from jax import lax
from jax.experimental import pallas as pl
from jax.experimental.pallas import tpu as pltpu
from jax.experimental.pallas import tpu_sc as plsc

N_SNPS = 100000
N_GENES = 9000
N_NODES = 90000
N_FILTERS = 8
BATCH = 16

NC = 2
NS = 16
NW = NC * NS

CW = 128
NODES_PAD = 90112
CHUNKS = NODES_PAD // (NW * CW)
NODES_PER_W = CHUNKS * CW

G_PAD = 9088
ROWS_PER_TILE = G_PAD // NS

_BLK_N = 2048
_GRID_N = (N_SNPS + _BLK_N - 1) // _BLK_N


def _scale_body(filt_ref, snp_ref, out_ref):
    fbar = jnp.sum(filt_ref[...], axis=0, keepdims=True) * (1.0 / N_FILTERS)
    out_ref[...] = (snp_ref[...] * fbar).T


def _scale_t(snp, filters):
    return pl.pallas_call(
        _scale_body,
        grid=(_GRID_N,),
        in_specs=[
            pl.BlockSpec((N_FILTERS, _BLK_N), lambda i: (0, i)),
            pl.BlockSpec((BATCH, _BLK_N), lambda i: (0, i)),
        ],
        out_specs=pl.BlockSpec((_BLK_N, BATCH), lambda i: (i, 0)),
        out_shape=jax.ShapeDtypeStruct((N_SNPS, BATCH), jnp.float32),
    )(filters, snp)


_sc_mesh = plsc.VectorSubcoreMesh(core_axis_name="c", subcore_axis_name="s")


@functools.partial(
    pl.kernel,
    mesh=_sc_mesh,
    compiler_params=pltpu.CompilerParams(use_tc_tiling_on_sc=False),
    out_type=jax.ShapeDtypeStruct((NC, G_PAD, BATCH), jnp.float32),
    scratch_types=[
        pltpu.VMEM((CHUNKS, CW), jnp.int32),
        pltpu.VMEM((CHUNKS, CW), jnp.int32),
        pltpu.VMEM((NODES_PER_W, BATCH), jnp.float32),
        pltpu.VMEM((ROWS_PER_TILE, BATCH), jnp.float32),
        pltpu.VMEM_SHARED((G_PAD, BATCH), jnp.float32),
        pltpu.SemaphoreType.DMA,
    ],
)
def _gather_segsum(a_hbm, idx_hbm, g_hbm, out_hbm,
                   idx_v, g_v, rows_v, zero_v, acc, sem):
    c = lax.axis_index("c")
    s = lax.axis_index("s")
    wid = c * NS + s

    pltpu.sync_copy(idx_hbm.at[wid], idx_v)
    pltpu.sync_copy(g_hbm.at[wid], g_v)

    def _z(i, carry):
        zero_v[i, :] = jnp.zeros((BATCH,), jnp.float32)
        return carry
    lax.fori_loop(0, ROWS_PER_TILE, _z, 0)
    pltpu.sync_copy(zero_v, acc.at[pl.ds(s * ROWS_PER_TILE, ROWS_PER_TILE)])
    plsc.subcore_barrier()

    copies = []
    for j in range(CHUNKS):
        copies.append(
            pltpu.async_copy(
                a_hbm.at[idx_v.at[j]], rows_v.at[pl.ds(j * CW, CW)], sem))
    for cp in copies:
        cp.wait()

    for j in range(CHUNKS):
        pltpu.sync_copy(
            rows_v.at[pl.ds(j * CW, CW)], acc.at[g_v.at[j]], add=True)
    plsc.subcore_barrier()

    pltpu.sync_copy(
        acc.at[pl.ds(s * ROWS_PER_TILE, ROWS_PER_TILE)],
        out_hbm.at[c, pl.ds(s * ROWS_PER_TILE, ROWS_PER_TILE)])


def _mlp_body(p_ref, w1_ref, b1_ref, g1_ref, be1_ref,
              w2_ref, b2_ref, g2_ref, be2_ref,
              w3_ref, b3_ref, g3_ref, be3_ref,
              w4_ref, b4_ref, out_ref):
    inv = float(1.0 / (1.0 + 1e-5) ** 0.5)
    psum = p_ref[0, :N_GENES, :] + p_ref[1, :N_GENES, :]
    h = jnp.dot(w1_ref[...], psum, preferred_element_type=jnp.float32)
    h = h + b1_ref[...]
    h = jnp.maximum(h * (g1_ref[...] * inv) + be1_ref[...], 0.0)
    h = jnp.dot(w2_ref[...], h, preferred_element_type=jnp.float32) + b2_ref[...]
    h = jnp.maximum(h * (g2_ref[...] * inv) + be2_ref[...], 0.0)
    h = jnp.dot(w3_ref[...], h, preferred_element_type=jnp.float32) + b3_ref[...]
    h = jnp.maximum(h * (g3_ref[...] * inv) + be3_ref[...], 0.0)
    out_ref[...] = (
        jnp.dot(w4_ref[...], h, preferred_element_type=jnp.float32)
        + b4_ref[...])


def _mlp(p, params):
    w1 = params['W1']
    b1 = params['b1'][:, None]
    g1 = params['g1'][:, None]
    be1 = params['be1'][:, None]
    w2 = params['W2']
    b2 = params['b2'][:, None]
    g2 = params['g2'][:, None]
    be2 = params['be2'][:, None]
    w3 = jnp.pad(params['W3'], ((0, 4), (0, 0)))
    b3 = jnp.pad(params['b3'], (0, 4))[:, None]
    g3 = jnp.pad(params['g3'], (0, 4))[:, None]
    be3 = jnp.pad(params['be3'], (0, 4))[:, None]
    w4 = jnp.pad(params['W4'], ((0, 7), (0, 4)))
    b4 = jnp.pad(params['b4'], (0, 7))[:, None]
    out = pl.pallas_call(
        _mlp_body,
        out_shape=jax.ShapeDtypeStruct((8, BATCH), jnp.float32),
    )(p, w1, b1, g1, be1, w2, b2, g2, be2, w3, b3, g3, be3, w4, b4)
    return out[0:1, :].T


def kernel(snp, snp_ids, g, filters, params):
    a = _scale_t(snp, filters)
    pad = NODES_PAD - N_NODES
    idx3 = jnp.pad(snp_ids.astype(jnp.int32), (0, pad)).reshape(NW, CHUNKS, CW)
    g3 = jnp.pad(g.astype(jnp.int32), (0, pad),
                 constant_values=G_PAD - 1).reshape(NW, CHUNKS, CW)
    partials = _gather_segsum(a, idx3, g3)
    return partials[0, :BATCH, :1]


# speedup vs baseline: 1.2799x; 1.2799x over previous
import functools

import jax
import jax.numpy as jnp
from jax import lax
from jax.experimental import pallas as pl
from jax.experimental.pallas import tpu as pltpu
from jax.experimental.pallas import tpu_sc as plsc

N_SNPS = 100000
N_GENES = 9000
N_NODES = 90000
N_FILTERS = 8
BATCH = 16

NC = 2
NS = 16
NW = NC * NS

CW = 128
NODES_PAD = 90112
CHUNKS = NODES_PAD // (NW * CW)
NODES_PER_W = CHUNKS * CW

SNP_PAD = 100352
COLS_PER_TILE = SNP_PAD // NW
COL_CHUNK = 1568
COL_CHUNKS = COLS_PER_TILE // COL_CHUNK
CGRPS = COL_CHUNK // 16

G_PAD = 9088
ROWS_PER_TILE = G_PAD // NS

_sc_mesh = plsc.VectorSubcoreMesh(core_axis_name="c", subcore_axis_name="s")
_sc_params = pltpu.CompilerParams(
    use_tc_tiling_on_sc=False, needs_layout_passes=False)


@functools.partial(
    pl.kernel,
    mesh=_sc_mesh,
    compiler_params=_sc_params,
    out_type=jax.ShapeDtypeStruct((SNP_PAD, BATCH), jnp.float32),
    scratch_types=[
        pltpu.VMEM((BATCH, COL_CHUNK), jnp.float32),
        pltpu.VMEM((N_FILTERS, COL_CHUNK), jnp.float32),
        pltpu.VMEM((COL_CHUNK, BATCH), jnp.float32),
    ],
)
def _build_table(snp_hbm, filt_hbm, a_hbm, snp_v, filt_v, trans_v):
    c = lax.axis_index("c")
    s = lax.axis_index("s")
    wid = c * NS + s
    lanes = lax.iota(jnp.int32, 16)
    for k in range(COL_CHUNKS):
        c0 = wid * COLS_PER_TILE + k * COL_CHUNK
        pltpu.sync_copy(snp_hbm.at[:, pl.ds(c0, COL_CHUNK)], snp_v)
        pltpu.sync_copy(filt_hbm.at[:, pl.ds(c0, COL_CHUNK)], filt_v)

        def _grp(gi, carry):
            n = gi * 16
            fb = filt_v[0, pl.ds(n, 16)]
            for f in range(1, N_FILTERS):
                fb = fb + filt_v[f, pl.ds(n, 16)]
            fb = fb * (1.0 / N_FILTERS)
            cols = lanes + n
            for b in range(BATCH):
                v = snp_v[b, pl.ds(n, 16)] * fb
                plsc.store_scatter(
                    trans_v, [cols, jnp.full((16,), b, jnp.int32)], v)
            return carry
        lax.fori_loop(0, CGRPS, _grp, 0)
        pltpu.sync_copy(trans_v, a_hbm.at[pl.ds(c0, COL_CHUNK)])


@functools.partial(
    pl.kernel,
    mesh=_sc_mesh,
    compiler_params=_sc_params,
    out_type=jax.ShapeDtypeStruct((NC, G_PAD, BATCH), jnp.float32),
    scratch_types=[
        pltpu.VMEM((CHUNKS, CW), jnp.int32),
        pltpu.VMEM((CHUNKS, CW), jnp.int32),
        pltpu.VMEM((NODES_PER_W, BATCH), jnp.float32),
        pltpu.VMEM_SHARED((G_PAD, BATCH), jnp.float32),
        pltpu.SemaphoreType.DMA,
    ],
)
def _gather_segsum(a_hbm, idx_hbm, g_hbm, out_hbm,
                   idx_v, g_v, rows_v, acc, sem):
    c = lax.axis_index("c")
    s = lax.axis_index("s")
    wid = c * NS + s

    pltpu.sync_copy(idx_hbm.at[wid], idx_v)
    pltpu.sync_copy(g_hbm.at[wid], g_v)

    def _z(i, carry):
        rows_v[i, :] = jnp.zeros((BATCH,), jnp.float32)
        return carry
    lax.fori_loop(0, ROWS_PER_TILE, _z, 0)
    pltpu.sync_copy(rows_v.at[pl.ds(0, ROWS_PER_TILE)],
                    acc.at[pl.ds(s * ROWS_PER_TILE, ROWS_PER_TILE)])
    plsc.subcore_barrier()

    copies = []
    for j in range(CHUNKS):
        copies.append(
            pltpu.async_copy(
                a_hbm.at[idx_v.at[j]], rows_v.at[pl.ds(j * CW, CW)], sem))
    for cp in copies:
        cp.wait()

    for j in range(CHUNKS):
        pltpu.sync_copy(
            rows_v.at[pl.ds(j * CW, CW)], acc.at[g_v.at[j]], add=True)
    plsc.subcore_barrier()

    pltpu.sync_copy(
        acc.at[pl.ds(s * ROWS_PER_TILE, ROWS_PER_TILE)],
        out_hbm.at[c, pl.ds(s * ROWS_PER_TILE, ROWS_PER_TILE)])


def _mlp_body(p_ref, w1_ref, b1_ref, g1_ref, be1_ref,
              w2_ref, b2_ref, g2_ref, be2_ref,
              w3_ref, b3_ref, g3_ref, be3_ref,
              w4_ref, b4_ref, out_ref):
    inv = float(1.0 / (1.0 + 1e-5) ** 0.5)
    psum = p_ref[0, :N_GENES, :] + p_ref[1, :N_GENES, :]
    h = jnp.dot(w1_ref[...], psum, preferred_element_type=jnp.float32)
    h = h + b1_ref[...]
    h = jnp.maximum(h * (g1_ref[...] * inv) + be1_ref[...], 0.0)
    h = jnp.dot(w2_ref[...], h, preferred_element_type=jnp.float32) + b2_ref[...]
    h = jnp.maximum(h * (g2_ref[...] * inv) + be2_ref[...], 0.0)
    h = jnp.dot(w3_ref[...], h, preferred_element_type=jnp.float32) + b3_ref[...]
    h = jnp.maximum(h * (g3_ref[...] * inv) + be3_ref[...], 0.0)
    out_ref[...] = (
        jnp.dot(w4_ref[...], h, preferred_element_type=jnp.float32)
        + b4_ref[...])


def _mlp(p, params):
    w1 = params['W1']
    b1 = params['b1'][:, None]
    g1 = params['g1'][:, None]
    be1 = params['be1'][:, None]
    w2 = params['W2']
    b2 = params['b2'][:, None]
    g2 = params['g2'][:, None]
    be2 = params['be2'][:, None]
    w3 = jnp.pad(params['W3'], ((0, 4), (0, 0)))
    b3 = jnp.pad(params['b3'], (0, 4))[:, None]
    g3 = jnp.pad(params['g3'], (0, 4))[:, None]
    be3 = jnp.pad(params['be3'], (0, 4))[:, None]
    w4 = jnp.pad(params['W4'], ((0, 7), (0, 4)))
    b4 = jnp.pad(params['b4'], (0, 7))[:, None]
    out = pl.pallas_call(
        _mlp_body,
        out_shape=jax.ShapeDtypeStruct((8, BATCH), jnp.float32),
    )(p, w1, b1, g1, be1, w2, b2, g2, be2, w3, b3, g3, be3, w4, b4)
    return out[0:1, :].T


def kernel(snp, snp_ids, g, filters, params):
    snp_p = jnp.pad(snp, ((0, 0), (0, SNP_PAD - N_SNPS)))
    filt_p = jnp.pad(filters, ((0, 0), (0, SNP_PAD - N_SNPS)))
    a = _build_table(snp_p, filt_p)
    pad = NODES_PAD - N_NODES
    idx3 = jnp.pad(snp_ids.astype(jnp.int32), (0, pad)).reshape(NW, CHUNKS, CW)
    g3 = jnp.pad(g.astype(jnp.int32), (0, pad),
                 constant_values=G_PAD - 1).reshape(NW, CHUNKS, CW)
    partials = _gather_segsum(a, idx3, g3)
    return _mlp(partials, params)


# speedup vs baseline: 1.3978x; 1.0921x over previous
import functools

import jax
import jax.numpy as jnp
from jax import lax
from jax.experimental import pallas as pl
from jax.experimental.pallas import tpu as pltpu
from jax.experimental.pallas import tpu_sc as plsc

N_SNPS = 100000
N_GENES = 9000
N_NODES = 90000
N_FILTERS = 8
BATCH = 16

NC = 2
NS = 16
NW = NC * NS

CW = 128
NODES_PAD = 90112
CHUNKS = NODES_PAD // (NW * CW)
NODES_PER_W = CHUNKS * CW

SNP_PAD = 100352
COLS_PER_TILE = SNP_PAD // NW
COL_CHUNK = 784
COL_CHUNKS = COLS_PER_TILE // COL_CHUNK
CGRPS = COL_CHUNK // 16
CFLAT = COL_CHUNK * BATCH

G_PAD = 9088
ROWS_PER_TILE = G_PAD // NS

_sc_mesh = plsc.VectorSubcoreMesh(core_axis_name="c", subcore_axis_name="s")
_sc_params = pltpu.CompilerParams(
    use_tc_tiling_on_sc=False, needs_layout_passes=False)


@functools.partial(
    pl.kernel,
    mesh=_sc_mesh,
    compiler_params=_sc_params,
    out_type=jax.ShapeDtypeStruct((SNP_PAD * BATCH,), jnp.float32),
    scratch_types=[
        pltpu.VMEM((2, BATCH, COL_CHUNK), jnp.float32),
        pltpu.VMEM((2, N_FILTERS, COL_CHUNK), jnp.float32),
        pltpu.VMEM((2, CFLAT), jnp.float32),
        pltpu.SemaphoreType.DMA,
        pltpu.SemaphoreType.DMA,
    ],
)
def _build_table(snp_hbm, filt_hbm, a_hbm, snp_v, filt_v, trans_v,
                 sem_in, sem_out):
    c = lax.axis_index("c")
    s = lax.axis_index("s")
    wid = c * NS + s
    base = wid * COLS_PER_TILE
    lanes = lax.iota(jnp.int32, 16)

    def _load(k, buf):
        c0 = base + k * COL_CHUNK
        return (
            pltpu.async_copy(
                snp_hbm.at[:, pl.ds(c0, COL_CHUNK)], snp_v.at[buf], sem_in),
            pltpu.async_copy(
                filt_hbm.at[:, pl.ds(c0, COL_CHUNK)], filt_v.at[buf], sem_in),
        )

    loads = {0: _load(0, 0)}
    writes = {}
    for k in range(COL_CHUNKS):
        buf = k % 2
        for cp in loads.pop(k):
            cp.wait()
        if k + 1 < COL_CHUNKS:
            loads[k + 1] = _load(k + 1, 1 - buf)
        if k >= 2:
            writes.pop(k - 2).wait()

        def _grp(gi, carry):
            n = gi * 16
            fb = filt_v[buf, 0, pl.ds(n, 16)]
            for f in range(1, N_FILTERS):
                fb = fb + filt_v[buf, f, pl.ds(n, 16)]
            fb = fb * (1.0 / N_FILTERS)
            flat = (lanes + n) * BATCH
            for b in range(BATCH):
                v = snp_v[buf, b, pl.ds(n, 16)] * fb
                plsc.store_scatter(trans_v.at[buf], [flat + b], v)
            return carry
        lax.fori_loop(0, CGRPS, _grp, 0)
        writes[k] = pltpu.async_copy(
            trans_v.at[buf],
            a_hbm.at[pl.ds((base + k * COL_CHUNK) * BATCH, CFLAT)], sem_out)
    for cp in writes.values():
        cp.wait()


@functools.partial(
    pl.kernel,
    mesh=_sc_mesh,
    compiler_params=_sc_params,
    out_type=jax.ShapeDtypeStruct((NC, G_PAD, BATCH), jnp.float32),
    scratch_types=[
        pltpu.VMEM((CHUNKS, CW), jnp.int32),
        pltpu.VMEM((CHUNKS, CW), jnp.int32),
        pltpu.VMEM((NODES_PER_W, BATCH), jnp.float32),
        pltpu.VMEM_SHARED((G_PAD, BATCH), jnp.float32),
        pltpu.SemaphoreType.DMA,
        pltpu.SemaphoreType.DMA,
    ],
)
def _gather_segsum(a_hbm, idx_hbm, g_hbm, out_hbm,
                   idx_v, g_v, rows_v, acc, sem, sem2):
    c = lax.axis_index("c")
    s = lax.axis_index("s")
    wid = c * NS + s

    pltpu.sync_copy(idx_hbm.at[wid], idx_v)
    pltpu.sync_copy(g_hbm.at[wid], g_v)

    def _z(i, carry):
        rows_v[i, :] = jnp.zeros((BATCH,), jnp.float32)
        return carry
    lax.fori_loop(0, ROWS_PER_TILE, _z, 0)
    pltpu.sync_copy(rows_v.at[pl.ds(0, ROWS_PER_TILE)],
                    acc.at[pl.ds(s * ROWS_PER_TILE, ROWS_PER_TILE)])
    plsc.subcore_barrier()

    gathers = []
    for j in range(CHUNKS):
        gathers.append(
            pltpu.async_copy(
                a_hbm.at[idx_v.at[j]], rows_v.at[pl.ds(j * CW, CW)], sem))
    scatters = []
    for j in range(CHUNKS):
        gathers[j].wait()
        scatters.append(
            pltpu.async_copy(
                rows_v.at[pl.ds(j * CW, CW)], acc.at[g_v.at[j]], sem2,
                add=True))
    for cp in scatters:
        cp.wait()
    plsc.subcore_barrier()

    pltpu.sync_copy(
        acc.at[pl.ds(s * ROWS_PER_TILE, ROWS_PER_TILE)],
        out_hbm.at[c, pl.ds(s * ROWS_PER_TILE, ROWS_PER_TILE)])


def _mlp_body(p_ref, w1_ref, pk_ref, out_ref):
    inv = float(1.0 / (1.0 + 1e-5) ** 0.5)
    pk = pk_ref[...]
    psum = p_ref[0, :N_GENES, :] + p_ref[1, :N_GENES, :]
    h = jnp.dot(w1_ref[...], psum, preferred_element_type=jnp.float32)
    h = h + pk[:, 0:1]
    h = jnp.maximum(h * (pk[:, 1:2] * inv) + pk[:, 2:3], 0.0)
    h = (jnp.dot(pk[0:16, 3:67], h, preferred_element_type=jnp.float32)
         + pk[0:16, 67:68])
    h = jnp.maximum(h * (pk[0:16, 68:69] * inv) + pk[0:16, 69:70], 0.0)
    h = (jnp.dot(pk[0:8, 70:86], h, preferred_element_type=jnp.float32)
         + pk[0:8, 86:87])
    h = jnp.maximum(h * (pk[0:8, 87:88] * inv) + pk[0:8, 88:89], 0.0)
    out_ref[...] = (
        jnp.dot(pk[0:8, 89:93], h[0:4, :], preferred_element_type=jnp.float32)
        + pk[0:8, 93:94])


def _pack_params(params):
    def col(v):
        return jnp.pad(v[:, None], ((0, 64 - v.shape[0]), (0, 0)))

    def blk(m):
        return jnp.pad(m, ((0, 64 - m.shape[0]), (0, 0)))

    return jnp.concatenate([
        col(params['b1']), col(params['g1']), col(params['be1']),
        blk(params['W2']),
        col(params['b2']), col(params['g2']), col(params['be2']),
        blk(params['W3']),
        col(params['b3']), col(params['g3']), col(params['be3']),
        blk(params['W4']),
        col(params['b4']),
    ], axis=1)


def _mlp(p, params):
    out = pl.pallas_call(
        _mlp_body,
        out_shape=jax.ShapeDtypeStruct((8, BATCH), jnp.float32),
    )(p, params['W1'], _pack_params(params))
    return out[0:1, :].T


def kernel(snp, snp_ids, g, filters, params):
    snp_p = jnp.pad(snp, ((0, 0), (0, SNP_PAD - N_SNPS)))
    filt_p = jnp.pad(filters, ((0, 0), (0, SNP_PAD - N_SNPS)))
    a = _build_table(snp_p, filt_p).reshape(SNP_PAD, BATCH)
    pad = NODES_PAD - N_NODES
    idx3 = jnp.pad(snp_ids.astype(jnp.int32), (0, pad)).reshape(NW, CHUNKS, CW)
    g3 = jnp.pad(g.astype(jnp.int32), (0, pad),
                 constant_values=G_PAD - 1).reshape(NW, CHUNKS, CW)
    partials = _gather_segsum(a, idx3, g3)
    return _mlp(partials, params)


# speedup vs baseline: 1.4233x; 1.0182x over previous
import functools

import jax
import jax.numpy as jnp
from jax import lax
from jax.experimental import pallas as pl
from jax.experimental.pallas import tpu as pltpu
from jax.experimental.pallas import tpu_sc as plsc

N_SNPS = 100000
N_GENES = 9000
N_NODES = 90000
N_FILTERS = 8
BATCH = 16

NC = 2
NS = 16
NW = NC * NS

CW = 128
NODES_PAD = 90112
CHUNKS = NODES_PAD // (NW * CW)
NODES_PER_W = CHUNKS * CW

SNP_PAD = 100352
COLS_PER_TILE = SNP_PAD // NW
COL_CHUNK = 784
COL_CHUNKS = COLS_PER_TILE // COL_CHUNK
CGRPS = COL_CHUNK // 16
CFLAT = COL_CHUNK * BATCH

G_PAD = 9088
ROWS_PER_TILE = G_PAD // NS

_sc_mesh = plsc.VectorSubcoreMesh(core_axis_name="c", subcore_axis_name="s")
_sc_params = pltpu.CompilerParams(
    use_tc_tiling_on_sc=False, needs_layout_passes=False)

def _fbar_body(filt_ref, out_ref):
    out_ref[...] = jnp.sum(filt_ref[...], axis=0, keepdims=True) * (
        1.0 / N_FILTERS)


def _fbar(filters):
    return pl.pallas_call(
        _fbar_body,
        out_shape=jax.ShapeDtypeStruct((1, SNP_PAD), jnp.float32),
    )(filters)


@functools.partial(
    pl.kernel,
    mesh=_sc_mesh,
    compiler_params=_sc_params,
    out_type=jax.ShapeDtypeStruct((SNP_PAD * BATCH,), jnp.float32),
    scratch_types=[
        pltpu.VMEM((2, BATCH, COL_CHUNK), jnp.float32),
        pltpu.VMEM((2, COL_CHUNK), jnp.float32),
        pltpu.VMEM((2, CFLAT), jnp.float32),
        pltpu.SemaphoreType.DMA,
        pltpu.SemaphoreType.DMA,
    ],
)
def _build_table(snp_hbm, fbar_hbm, a_hbm, snp_v, fb_v, trans_v,
                 sem_in, sem_out):
    c = lax.axis_index("c")
    s = lax.axis_index("s")
    wid = c * NS + s
    base = wid * COLS_PER_TILE
    lanes = lax.iota(jnp.int32, 16)

    def _load(k, buf):
        return (
            pltpu.async_copy(
                snp_hbm.at[:, pl.ds(base + k * COL_CHUNK, COL_CHUNK)],
                snp_v.at[buf], sem_in),
            pltpu.async_copy(
                fbar_hbm.at[0, pl.ds(base + k * COL_CHUNK, COL_CHUNK)],
                fb_v.at[buf], sem_in),
        )

    loads = {0: _load(0, 0)}
    writes = {}
    for k in range(COL_CHUNKS):
        buf = k % 2
        for cp in loads.pop(k):
            cp.wait()
        if k + 1 < COL_CHUNKS:
            loads[k + 1] = _load(k + 1, 1 - buf)
        if k >= 2:
            writes.pop(k - 2).wait()

        def _grp(gi, carry):
            n = gi * 16
            fb = fb_v[buf, pl.ds(n, 16)]
            flat = (lanes + n) * BATCH
            for b in range(BATCH):
                v = snp_v[buf, b, pl.ds(n, 16)] * fb
                plsc.store_scatter(trans_v.at[buf], [flat + b], v)
            return carry
        lax.fori_loop(0, CGRPS, _grp, 0)
        writes[k] = pltpu.async_copy(
            trans_v.at[buf],
            a_hbm.at[pl.ds((base + k * COL_CHUNK) * BATCH, CFLAT)], sem_out)
    for cp in writes.values():
        cp.wait()


@functools.partial(
    pl.kernel,
    mesh=_sc_mesh,
    compiler_params=_sc_params,
    out_type=jax.ShapeDtypeStruct((NC, G_PAD, BATCH), jnp.float32),
    scratch_types=[
        pltpu.VMEM((CHUNKS, CW), jnp.int32),
        pltpu.VMEM((CHUNKS, CW), jnp.int32),
        pltpu.VMEM((NODES_PER_W, BATCH), jnp.float32),
        pltpu.VMEM_SHARED((G_PAD, BATCH), jnp.float32),
        pltpu.SemaphoreType.DMA,
        pltpu.SemaphoreType.DMA,
    ],
)
def _gather_segsum(a_hbm, idx_hbm, g_hbm, out_hbm,
                   idx_v, g_v, rows_v, acc, sem, sem2):
    c = lax.axis_index("c")
    s = lax.axis_index("s")
    wid = c * NS + s

    pltpu.sync_copy(idx_hbm.at[wid], idx_v)
    pltpu.sync_copy(g_hbm.at[wid], g_v)

    def _z(i, carry):
        rows_v[i, :] = jnp.zeros((BATCH,), jnp.float32)
        return carry
    lax.fori_loop(0, ROWS_PER_TILE, _z, 0)
    pltpu.sync_copy(rows_v.at[pl.ds(0, ROWS_PER_TILE)],
                    acc.at[pl.ds(s * ROWS_PER_TILE, ROWS_PER_TILE)])
    plsc.subcore_barrier()

    gathers = []
    for j in range(CHUNKS):
        gathers.append(
            pltpu.async_copy(
                a_hbm.at[idx_v.at[j]], rows_v.at[pl.ds(j * CW, CW)], sem))
    scatters = []
    for j in range(CHUNKS):
        gathers[j].wait()
        scatters.append(
            pltpu.async_copy(
                rows_v.at[pl.ds(j * CW, CW)], acc.at[g_v.at[j]], sem2,
                add=True))
    for cp in scatters:
        cp.wait()
    plsc.subcore_barrier()

    pltpu.sync_copy(
        acc.at[pl.ds(s * ROWS_PER_TILE, ROWS_PER_TILE)],
        out_hbm.at[c, pl.ds(s * ROWS_PER_TILE, ROWS_PER_TILE)])


def _mlp_body(p_ref, w1_ref, pk_ref, out_ref):
    inv = float(1.0 / (1.0 + 1e-5) ** 0.5)
    pk = pk_ref[...]
    psum = p_ref[0, :N_GENES, :] + p_ref[1, :N_GENES, :]
    h = jnp.dot(w1_ref[...], psum, preferred_element_type=jnp.float32)
    h = h + pk[:, 0:1]
    h = jnp.maximum(h * (pk[:, 1:2] * inv) + pk[:, 2:3], 0.0)
    h = (jnp.dot(pk[0:16, 3:67], h, preferred_element_type=jnp.float32)
         + pk[0:16, 67:68])
    h = jnp.maximum(h * (pk[0:16, 68:69] * inv) + pk[0:16, 69:70], 0.0)
    h = (jnp.dot(pk[0:8, 70:86], h, preferred_element_type=jnp.float32)
         + pk[0:8, 86:87])
    h = jnp.maximum(h * (pk[0:8, 87:88] * inv) + pk[0:8, 88:89], 0.0)
    out_ref[...] = (
        jnp.dot(pk[0:8, 89:93], h[0:4, :], preferred_element_type=jnp.float32)
        + pk[0:8, 93:94])


def _pack_params(params):
    def col(v):
        return jnp.pad(v[:, None], ((0, 64 - v.shape[0]), (0, 0)))

    def blk(m):
        return jnp.pad(m, ((0, 64 - m.shape[0]), (0, 0)))

    return jnp.concatenate([
        col(params['b1']), col(params['g1']), col(params['be1']),
        blk(params['W2']),
        col(params['b2']), col(params['g2']), col(params['be2']),
        blk(params['W3']),
        col(params['b3']), col(params['g3']), col(params['be3']),
        blk(params['W4']),
        col(params['b4']),
    ], axis=1)


def _mlp(p, params):
    out = pl.pallas_call(
        _mlp_body,
        out_shape=jax.ShapeDtypeStruct((8, BATCH), jnp.float32),
    )(p, params['W1'], _pack_params(params))
    return out[0:1, :].T


def kernel(snp, snp_ids, g, filters, params):
    snp_p = jnp.pad(snp, ((0, 0), (0, SNP_PAD - N_SNPS)))
    filt_p = jnp.pad(filters, ((0, 0), (0, SNP_PAD - N_SNPS)))
    a = _build_table(snp_p, _fbar(filt_p)).reshape(SNP_PAD, BATCH)
    pad = NODES_PAD - N_NODES
    idx3 = jnp.pad(snp_ids.astype(jnp.int32), (0, pad)).reshape(NW, CHUNKS, CW)
    g3 = jnp.pad(g.astype(jnp.int32), (0, pad),
                 constant_values=G_PAD - 1).reshape(NW, CHUNKS, CW)
    partials = _gather_segsum(a, idx3, g3)
    return _mlp(partials, params)


# speedup vs baseline: 1.5691x; 1.1024x over previous
import functools

import jax
import jax.numpy as jnp
from jax import lax
from jax.experimental import pallas as pl
from jax.experimental.pallas import tpu as pltpu
from jax.experimental.pallas import tpu_sc as plsc

N_SNPS = 100000
N_GENES = 9000
N_NODES = 90000
N_FILTERS = 8
BATCH = 16

NC = 2
NS = 16
NW = NC * NS

CW = 128
NODES_PAD = 90112
CHUNKS = NODES_PAD // (NW * CW)
NODES_PER_W = CHUNKS * CW

SNP_PAD = 100352
COLS_PER_TILE = SNP_PAD // NW
COL_CHUNK = 784
COL_CHUNKS = COLS_PER_TILE // COL_CHUNK
CGRPS = COL_CHUNK // 16
CFLAT = COL_CHUNK * BATCH

G_PAD = 9088
ROWS_PER_TILE = G_PAD // NS

_sc_mesh = plsc.VectorSubcoreMesh(core_axis_name="c", subcore_axis_name="s")
_sc_params = pltpu.CompilerParams(
    use_tc_tiling_on_sc=False, needs_layout_passes=False)

def _fbar_body(filt_ref, out_ref):
    out_ref[...] = jnp.sum(filt_ref[...], axis=0, keepdims=True) * (
        1.0 / N_FILTERS)


def _fbar(filters):
    return pl.pallas_call(
        _fbar_body,
        out_shape=jax.ShapeDtypeStruct((1, SNP_PAD), jnp.float32),
    )(filters)


@functools.partial(
    pl.kernel,
    mesh=_sc_mesh,
    compiler_params=_sc_params,
    out_type=jax.ShapeDtypeStruct((SNP_PAD * BATCH,), jnp.float32),
    scratch_types=[
        pltpu.VMEM((2, BATCH, COL_CHUNK), jnp.float32),
        pltpu.VMEM((2, COL_CHUNK), jnp.float32),
        pltpu.VMEM((2, CFLAT), jnp.float32),
        pltpu.SemaphoreType.DMA,
        pltpu.SemaphoreType.DMA,
    ],
)
def _build_table(snp_hbm, fbar_hbm, a_hbm, snp_v, fb_v, trans_v,
                 sem_in, sem_out):
    c = lax.axis_index("c")
    s = lax.axis_index("s")
    wid = c * NS + s
    base = wid * COLS_PER_TILE
    lanes = lax.iota(jnp.int32, 16)

    def _load(k, buf):
        return (
            pltpu.async_copy(
                snp_hbm.at[:, pl.ds(base + k * COL_CHUNK, COL_CHUNK)],
                snp_v.at[buf], sem_in),
            pltpu.async_copy(
                fbar_hbm.at[0, pl.ds(base + k * COL_CHUNK, COL_CHUNK)],
                fb_v.at[buf], sem_in),
        )

    loads = {0: _load(0, 0)}
    writes = {}
    for k in range(COL_CHUNKS):
        buf = k % 2
        for cp in loads.pop(k):
            cp.wait()
        if k + 1 < COL_CHUNKS:
            loads[k + 1] = _load(k + 1, 1 - buf)
        if k >= 2:
            writes.pop(k - 2).wait()

        def _grp(gi, carry):
            n = gi * 16
            fb = fb_v[buf, pl.ds(n, 16)]
            flat = (lanes + n) * BATCH
            for b in range(BATCH):
                v = snp_v[buf, b, pl.ds(n, 16)] * fb
                plsc.store_scatter(trans_v.at[buf], [flat + b], v)
            return carry
        lax.fori_loop(0, CGRPS, _grp, 0)
        writes[k] = pltpu.async_copy(
            trans_v.at[buf],
            a_hbm.at[pl.ds((base + k * COL_CHUNK) * BATCH, CFLAT)], sem_out)
    for cp in writes.values():
        cp.wait()


@functools.partial(
    pl.kernel,
    mesh=_sc_mesh,
    compiler_params=_sc_params,
    out_type=jax.ShapeDtypeStruct((NC, G_PAD, BATCH), jnp.float32),
    scratch_types=[
        pltpu.VMEM((CHUNKS, CW), jnp.int32),
        pltpu.VMEM((CHUNKS, CW), jnp.int32),
        pltpu.VMEM((NODES_PER_W, BATCH), jnp.float32),
        pltpu.VMEM((ROWS_PER_TILE, BATCH), jnp.float32),
        pltpu.VMEM_SHARED((G_PAD, BATCH), jnp.float32),
        pltpu.SemaphoreType.DMA,
        pltpu.SemaphoreType.DMA,
    ],
)
def _gather_segsum(a_hbm, idx_hbm, g_hbm, out_hbm,
                   idx_v, g_v, rows_v, zero_v, acc, sem, sem2):
    c = lax.axis_index("c")
    s = lax.axis_index("s")
    wid = c * NS + s

    pltpu.sync_copy(idx_hbm.at[wid], idx_v)
    gathers = []
    for j in range(CHUNKS):
        gathers.append(
            pltpu.async_copy(
                a_hbm.at[idx_v.at[j]], rows_v.at[pl.ds(j * CW, CW)], sem))
    pltpu.sync_copy(g_hbm.at[wid], g_v)

    def _z(i, carry):
        zero_v[i, :] = jnp.zeros((BATCH,), jnp.float32)
        return carry
    lax.fori_loop(0, ROWS_PER_TILE, _z, 0)
    pltpu.sync_copy(zero_v, acc.at[pl.ds(s * ROWS_PER_TILE, ROWS_PER_TILE)])
    plsc.subcore_barrier()

    scatters = []
    for j in range(CHUNKS):
        gathers[j].wait()
        scatters.append(
            pltpu.async_copy(
                rows_v.at[pl.ds(j * CW, CW)], acc.at[g_v.at[j]], sem2,
                add=True))
    for cp in scatters:
        cp.wait()
    plsc.subcore_barrier()

    pltpu.sync_copy(
        acc.at[pl.ds(s * ROWS_PER_TILE, ROWS_PER_TILE)],
        out_hbm.at[c, pl.ds(s * ROWS_PER_TILE, ROWS_PER_TILE)])


def _mlp_body(p_ref, w1_ref, pk_ref, out_ref):
    inv = float(1.0 / (1.0 + 1e-5) ** 0.5)
    pk = pk_ref[...]
    p2 = p_ref[0] + p_ref[1]
    h = None
    for j in range(8):
        hj = jnp.dot(w1_ref[:, j, :], p2, preferred_element_type=jnp.float32)
        part = hj[:, 16 * j:16 * (j + 1)]
        h = part if h is None else h + part
    h = h + pk[:, 0:1]
    h = jnp.maximum(h * (pk[:, 1:2] * inv) + pk[:, 2:3], 0.0)
    h = (jnp.dot(pk[0:16, 3:67], h, preferred_element_type=jnp.float32)
         + pk[0:16, 67:68])
    h = jnp.maximum(h * (pk[0:16, 68:69] * inv) + pk[0:16, 69:70], 0.0)
    h = (jnp.dot(pk[0:8, 70:86], h, preferred_element_type=jnp.float32)
         + pk[0:8, 86:87])
    h = jnp.maximum(h * (pk[0:8, 87:88] * inv) + pk[0:8, 88:89], 0.0)
    out_ref[...] = (
        jnp.dot(pk[0:8, 89:93], h[0:4, :], preferred_element_type=jnp.float32)
        + pk[0:8, 93:94])


def _pack_params(params):
    def col(v):
        return jnp.pad(v[:, None], ((0, 64 - v.shape[0]), (0, 0)))

    def blk(m):
        return jnp.pad(m, ((0, 64 - m.shape[0]), (0, 0)))

    return jnp.concatenate([
        col(params['b1']), col(params['g1']), col(params['be1']),
        blk(params['W2']),
        col(params['b2']), col(params['g2']), col(params['be2']),
        blk(params['W3']),
        col(params['b3']), col(params['g3']), col(params['be3']),
        blk(params['W4']),
        col(params['b4']),
    ], axis=1)


def _mlp(p, params):
    p2 = p.reshape(NC, G_PAD // 8, 128)
    w1r = jnp.pad(params['W1'], ((0, 0), (0, G_PAD - N_GENES)))
    w1r = w1r.reshape(64, G_PAD // 8, 8).transpose(0, 2, 1)
    out = pl.pallas_call(
        _mlp_body,
        out_shape=jax.ShapeDtypeStruct((8, BATCH), jnp.float32),
    )(p2, w1r, _pack_params(params))
    return out[0:1, :].T


def kernel(snp, snp_ids, g, filters, params):
    snp_p = jnp.pad(snp, ((0, 0), (0, SNP_PAD - N_SNPS)))
    filt_p = jnp.pad(filters, ((0, 0), (0, SNP_PAD - N_SNPS)))
    a = _build_table(snp_p, _fbar(filt_p)).reshape(SNP_PAD, BATCH)
    pad = NODES_PAD - N_NODES
    idx3 = jnp.pad(snp_ids.astype(jnp.int32), (0, pad)).reshape(NW, CHUNKS, CW)
    g3 = jnp.pad(g.astype(jnp.int32), (0, pad),
                 constant_values=G_PAD - 1).reshape(NW, CHUNKS, CW)
    partials = _gather_segsum(a, idx3, g3)
    return _mlp(partials, params)


# speedup vs baseline: 1.5966x; 1.0176x over previous
import functools

import jax
import jax.numpy as jnp
from jax import lax
from jax.experimental import pallas as pl
from jax.experimental.pallas import tpu as pltpu
from jax.experimental.pallas import tpu_sc as plsc

N_SNPS = 100000
N_GENES = 9000
N_NODES = 90000
N_FILTERS = 8
BATCH = 16

NC = 2
NS = 16
NW = NC * NS

CW = 128
NODES_PAD = 90112
CHUNKS = NODES_PAD // (NW * CW)
NODES_PER_W = CHUNKS * CW

SNP_PAD = 100352
COLS_PER_TILE = SNP_PAD // NW
COL_CHUNK = 784
COL_CHUNKS = COLS_PER_TILE // COL_CHUNK
CGRPS = COL_CHUNK // 16
CFLAT = COL_CHUNK * BATCH

G_PAD = 9088
ROWS_PER_TILE = G_PAD // NS

_sc_mesh = plsc.VectorSubcoreMesh(core_axis_name="c", subcore_axis_name="s")
_sc_params = pltpu.CompilerParams(
    use_tc_tiling_on_sc=False, needs_layout_passes=False)

def _fbar_body(filt_ref, out_ref):
    out_ref[...] = jnp.zeros((1, SNP_PAD), jnp.float32)
    out_ref[:, pl.ds(0, N_SNPS)] = jnp.sum(
        filt_ref[...], axis=0, keepdims=True) * (1.0 / N_FILTERS)


def _fbar(filters):
    return pl.pallas_call(
        _fbar_body,
        out_shape=jax.ShapeDtypeStruct((1, SNP_PAD), jnp.float32),
    )(filters)


@functools.partial(
    pl.kernel,
    mesh=_sc_mesh,
    compiler_params=_sc_params,
    out_type=jax.ShapeDtypeStruct((SNP_PAD * BATCH,), jnp.float32),
    scratch_types=[
        pltpu.VMEM((2, BATCH, COL_CHUNK), jnp.float32),
        pltpu.VMEM((2, COL_CHUNK), jnp.float32),
        pltpu.VMEM((2, CFLAT), jnp.float32),
        pltpu.SemaphoreType.DMA,
        pltpu.SemaphoreType.DMA,
    ],
)
def _build_table(snp_hbm, fbar_hbm, a_hbm, snp_v, fb_v, trans_v,
                 sem_in, sem_out):
    c = lax.axis_index("c")
    s = lax.axis_index("s")
    wid = c * NS + s
    base = wid * COLS_PER_TILE
    lanes = lax.iota(jnp.int32, 16)

    def _load(k, buf):
        return (
            pltpu.async_copy(
                snp_hbm.at[:, pl.ds(base + k * COL_CHUNK, COL_CHUNK)],
                snp_v.at[buf], sem_in),
            pltpu.async_copy(
                fbar_hbm.at[0, pl.ds(base + k * COL_CHUNK, COL_CHUNK)],
                fb_v.at[buf], sem_in),
        )

    loads = {0: _load(0, 0)}
    writes = {}
    for k in range(COL_CHUNKS):
        buf = k % 2
        for cp in loads.pop(k):
            cp.wait()
        if k + 1 < COL_CHUNKS:
            loads[k + 1] = _load(k + 1, 1 - buf)
        if k >= 2:
            writes.pop(k - 2).wait()

        def _grp(gi, carry):
            n = gi * 16
            fb = fb_v[buf, pl.ds(n, 16)]
            flat = (lanes + n) * BATCH
            for b in range(BATCH):
                v = snp_v[buf, b, pl.ds(n, 16)] * fb
                plsc.store_scatter(trans_v.at[buf], [flat + b], v)
            return carry
        lax.fori_loop(0, CGRPS, _grp, 0)
        writes[k] = pltpu.async_copy(
            trans_v.at[buf],
            a_hbm.at[pl.ds((base + k * COL_CHUNK) * BATCH, CFLAT)], sem_out)
    for cp in writes.values():
        cp.wait()


@functools.partial(
    pl.kernel,
    mesh=_sc_mesh,
    compiler_params=_sc_params,
    out_type=jax.ShapeDtypeStruct((NC, G_PAD, BATCH), jnp.float32),
    scratch_types=[
        pltpu.VMEM((CHUNKS, CW), jnp.int32),
        pltpu.VMEM((CHUNKS, CW), jnp.int32),
        pltpu.VMEM((NODES_PER_W, BATCH), jnp.float32),
        pltpu.VMEM((ROWS_PER_TILE, BATCH), jnp.float32),
        pltpu.VMEM_SHARED((G_PAD, BATCH), jnp.float32),
        pltpu.SemaphoreType.DMA,
        pltpu.SemaphoreType.DMA,
    ],
)
def _gather_segsum(a_hbm, idx_hbm, g_hbm, out_hbm,
                   idx_v, g_v, rows_v, zero_v, acc, sem, sem2):
    c = lax.axis_index("c")
    s = lax.axis_index("s")
    wid = c * NS + s

    pltpu.sync_copy(idx_hbm.at[wid], idx_v)
    gathers = []
    for j in range(CHUNKS):
        gathers.append(
            pltpu.async_copy(
                a_hbm.at[idx_v.at[j]], rows_v.at[pl.ds(j * CW, CW)], sem))
    pltpu.sync_copy(g_hbm.at[wid], g_v)

    def _z(i, carry):
        zero_v[i, :] = jnp.zeros((BATCH,), jnp.float32)
        return carry
    lax.fori_loop(0, ROWS_PER_TILE, _z, 0)
    pltpu.sync_copy(zero_v, acc.at[pl.ds(s * ROWS_PER_TILE, ROWS_PER_TILE)])
    plsc.subcore_barrier()

    scatters = []
    for j in range(CHUNKS):
        gathers[j].wait()
        scatters.append(
            pltpu.async_copy(
                rows_v.at[pl.ds(j * CW, CW)], acc.at[g_v.at[j]], sem2,
                add=True))
    for cp in scatters:
        cp.wait()
    plsc.subcore_barrier()

    pltpu.sync_copy(
        acc.at[pl.ds(s * ROWS_PER_TILE, ROWS_PER_TILE)],
        out_hbm.at[c, pl.ds(s * ROWS_PER_TILE, ROWS_PER_TILE)])


def _mlp_body(p_ref, w1_ref, pk_ref, out_ref):
    inv = float(1.0 / (1.0 + 1e-5) ** 0.5)
    pk = pk_ref[...]
    p2 = p_ref[0] + p_ref[1]
    h = None
    for j in range(8):
        hj = jnp.dot(w1_ref[:, j, :], p2, preferred_element_type=jnp.float32)
        part = hj[:, 16 * j:16 * (j + 1)]
        h = part if h is None else h + part
    h = h + pk[:, 0:1]
    h = jnp.maximum(h * (pk[:, 1:2] * inv) + pk[:, 2:3], 0.0)
    h = (jnp.dot(pk[0:16, 3:67], h, preferred_element_type=jnp.float32)
         + pk[0:16, 67:68])
    h = jnp.maximum(h * (pk[0:16, 68:69] * inv) + pk[0:16, 69:70], 0.0)
    h = (jnp.dot(pk[0:8, 70:86], h, preferred_element_type=jnp.float32)
         + pk[0:8, 86:87])
    h = jnp.maximum(h * (pk[0:8, 87:88] * inv) + pk[0:8, 88:89], 0.0)
    out_ref[...] = (
        jnp.dot(pk[0:8, 89:93], h[0:4, :], preferred_element_type=jnp.float32)
        + pk[0:8, 93:94])


def _pack_params(params):
    def col(v):
        return jnp.pad(v[:, None], ((0, 64 - v.shape[0]), (0, 0)))

    def blk(m):
        return jnp.pad(m, ((0, 64 - m.shape[0]), (0, 0)))

    return jnp.concatenate([
        col(params['b1']), col(params['g1']), col(params['be1']),
        blk(params['W2']),
        col(params['b2']), col(params['g2']), col(params['be2']),
        blk(params['W3']),
        col(params['b3']), col(params['g3']), col(params['be3']),
        blk(params['W4']),
        col(params['b4']),
    ], axis=1)


def _mlp(p, params):
    p2 = p.reshape(NC, G_PAD // 8, 128)
    w1r = jnp.pad(params['W1'], ((0, 0), (0, G_PAD - N_GENES)))
    w1r = w1r.reshape(64, G_PAD // 8, 8).transpose(0, 2, 1)
    out = pl.pallas_call(
        _mlp_body,
        out_shape=jax.ShapeDtypeStruct((8, BATCH), jnp.float32),
    )(p2, w1r, _pack_params(params))
    return out[0:1, :].T


def kernel(snp, snp_ids, g, filters, params):
    snp_p = jnp.pad(snp, ((0, 0), (0, SNP_PAD - N_SNPS)))
    a = _build_table(snp_p, _fbar(filters)).reshape(SNP_PAD, BATCH)
    pad = NODES_PAD - N_NODES
    idx3 = jnp.pad(snp_ids.astype(jnp.int32), (0, pad)).reshape(NW, CHUNKS, CW)
    g3 = jnp.pad(g.astype(jnp.int32), (0, pad),
                 constant_values=G_PAD - 1).reshape(NW, CHUNKS, CW)
    partials = _gather_segsum(a, idx3, g3)
    return _mlp(partials, params)
